# baseline (speedup 1.0000x reference)
import functools

import jax
import jax.numpy as jnp
import numpy as np
from jax import lax
from jax.experimental import pallas as pl
from jax.experimental.pallas import tpu as pltpu
from jax.experimental.pallas import tpu_sc as plsc

N = 10000
E = 320000
NC = 2
NS = 16
C = 128
EW = -(-E // (NS * 2 * C)) * 2 * C
NCH = EW // C
NP = 10240
ROWS_PER_TILE = NP // NS
ZROWS = 128

_PERM32 = np.empty(32, dtype=np.int32)
_PERM32[0::2] = np.arange(16)
_PERM32[1::2] = np.arange(16) + 16


def _packed_perm(d):
    return (np.arange(0, d, 32)[:, None] + _PERM32[None, :]).reshape(d)


def _spmm_sc(dense2w, colp, rowp, valp, d2):
    d2b = d2 // 2
    mesh = plsc.VectorSubcoreMesh(core_axis_name="c", subcore_axis_name="s")

    @functools.partial(
        pl.kernel,
        out_type=jax.ShapeDtypeStruct((NC, NP, d2), jnp.float32),
        mesh=mesh,
        scratch_types=[
            pltpu.VMEM((NCH, C), jnp.int32),
            pltpu.VMEM((NCH, C), jnp.int32),
            pltpu.VMEM((NCH, C), jnp.float32),
            pltpu.VMEM((C, d2b), jnp.int32),
            pltpu.VMEM((C, d2b), jnp.int32),
            pltpu.VMEM((C, d2), jnp.float32),
            pltpu.VMEM((C, d2), jnp.float32),
            pltpu.VMEM_SHARED((NP, d2), jnp.float32),
            pltpu.SemaphoreType.DMA,
            pltpu.SemaphoreType.DMA,
            pltpu.SemaphoreType.DMA,
            pltpu.SemaphoreType.DMA,
        ],
        compiler_params=pltpu.CompilerParams(use_tc_tiling_on_sc=False, needs_layout_passes=False),
    )
    def k(dense_hbm, col_hbm, row_hbm, val_hbm, out_hbm,
          colv, rowv, valv, gbuf0, gbuf1, fbuf0, fbuf1, acc,
          gsem0, gsem1, ssem0, ssem1):
        cid = lax.axis_index("c")
        sid = lax.axis_index("s")
        gbufs = (gbuf0, gbuf1)
        fbufs = (fbuf0, fbuf1)
        gsems = (gsem0, gsem1)
        ssems = (ssem0, ssem1)

        pltpu.sync_copy(col_hbm.at[sid], colv)
        pltpu.sync_copy(row_hbm.at[sid], rowv)
        pltpu.sync_copy(val_hbm.at[sid], valv)

        off = cid * N

        def shift(j, c2):
            for kk in range(C // 16):
                sl = pl.ds(kk * 16, 16)
                colv[j, sl] = colv[j, sl] + off
            return c2

        lax.fori_loop(0, NCH, shift, 0)

        zero = jnp.zeros((16,), jnp.float32)

        def zrow(i, carry):
            for kk in range(d2 // 16):
                fbuf0[i, pl.ds(kk * 16, 16)] = zero
            return carry

        lax.fori_loop(0, C, zrow, 0)
        for b in range(ROWS_PER_TILE // C):
            pltpu.sync_copy(
                fbuf0, acc.at[pl.ds(sid * ROWS_PER_TILE + b * C, C)])
        plsc.subcore_barrier()

        hi_mask = jnp.full((16,), -65536, jnp.int32)

        def scale(gb, fb, j):
            def grp(t, c2):
                v16 = valv[j, pl.ds(t * 16, 16)]
                for e16 in range(16):
                    v = v16[e16]
                    e = t * 16 + e16
                    for kk in range(d2b // 16):
                        w = gb[e, pl.ds(kk * 16, 16)]
                        lo = plsc.bitcast(w << 16, jnp.float32)
                        hi = plsc.bitcast(w & hi_mask, jnp.float32)
                        fb[e, pl.ds(kk * 32, 16)] = lo * v
                        fb[e, pl.ds(kk * 32 + 16, 16)] = hi * v
                return c2

            lax.fori_loop(0, C // 16, grp, 0)

        pltpu.async_copy(dense_hbm.at[colv.at[0]], gbuf0, gsem0)
        pltpu.async_copy(dense_hbm.at[colv.at[1]], gbuf1, gsem1)

        def body(jj, carry):
            for s in range(2):
                j = 2 * jj + s
                pltpu.make_async_copy(
                    dense_hbm.at[colv.at[j]], gbufs[s], gsems[s]).wait()

                @pl.when(jj >= 1)
                def _wait_scatter():
                    pltpu.make_async_copy(
                        fbufs[s], acc.at[rowv.at[j - 2]], ssems[s]).wait()

                scale(gbufs[s], fbufs[s], j)

                @pl.when(jj < NCH // 2 - 1)
                def _next_gather():
                    pltpu.async_copy(
                        dense_hbm.at[colv.at[j + 2]], gbufs[s], gsems[s])

                pltpu.async_copy(
                    fbufs[s], acc.at[rowv.at[j]], ssems[s], add=True)
            return carry

        lax.fori_loop(0, NCH // 2, body, 0)
        pltpu.make_async_copy(fbuf0, acc.at[rowv.at[NCH - 2]], ssem0).wait()
        pltpu.make_async_copy(fbuf1, acc.at[rowv.at[NCH - 1]], ssem1).wait()

        plsc.subcore_barrier()
        pltpu.sync_copy(acc.at[pl.ds(sid * ROWS_PER_TILE, ROWS_PER_TILE)],
                        out_hbm.at[cid, pl.ds(sid * ROWS_PER_TILE, ROWS_PER_TILE)])

    return k(dense2w, colp, rowp, valp)


_BN = 1000


def _mm_body(x_ref, w_ref, o_ref):
    o_ref[...] = jnp.dot(x_ref[...], w_ref[0],
                         preferred_element_type=jnp.float32
                         ).astype(jnp.bfloat16)


def _matmul_split_tc(x, ws):
    n, kd = x.shape
    m2 = ws.shape[2]
    return pl.pallas_call(
        _mm_body,
        grid=(NC, n // _BN),
        in_specs=[
            pl.BlockSpec((_BN, kd), lambda c, i: (i, 0)),
            pl.BlockSpec((1, kd, m2), lambda c, i: (c, 0, 0)),
        ],
        out_specs=pl.BlockSpec(
            (_BN, m2), lambda c, i: (c * (n // _BN) + i, 0)),
        out_shape=jax.ShapeDtypeStruct((NC * n, m2), jnp.bfloat16),
    )(x, ws)


def _fuse2_body(p_ref, w_ref, o_ref):
    h0 = jnp.maximum(p_ref[0], 0.0)
    h1 = jnp.maximum(p_ref[1], 0.0)
    w = w_ref[0]
    kd2 = p_ref.shape[2]
    o_ref[...] = (
        jnp.dot(h0, w[:kd2], preferred_element_type=jnp.float32)
        + jnp.dot(h1, w[kd2:], preferred_element_type=jnp.float32)
    ).astype(jnp.bfloat16)


def _fuse2_tc(p, ws):
    _, _, kd2 = p.shape
    m2 = ws.shape[2]
    n = N
    return pl.pallas_call(
        _fuse2_body,
        grid=(NC, n // _BN),
        in_specs=[
            pl.BlockSpec((NC, _BN, kd2), lambda c, i: (0, i, 0)),
            pl.BlockSpec((1, 2 * kd2, m2), lambda c, i: (c, 0, 0)),
        ],
        out_specs=pl.BlockSpec(
            (_BN, m2), lambda c, i: (c * (n // _BN) + i, 0)),
        out_shape=jax.ShapeDtypeStruct((NC * n, m2), jnp.bfloat16),
    )(p, ws)


def _lsm_body(q_ref, o_ref):
    q0 = q_ref[0]
    q1 = q_ref[1]
    m = jnp.maximum(jnp.max(q0, axis=1, keepdims=True),
                    jnp.max(q1, axis=1, keepdims=True))
    ssum = (jnp.sum(jnp.exp(q0 - m), axis=1, keepdims=True)
            + jnp.sum(jnp.exp(q1 - m), axis=1, keepdims=True))
    lse = jnp.log(ssum) + m
    m2 = q0.shape[1]
    o_ref[:, :m2] = q0 - lse
    o_ref[:, m2:] = q1 - lse


def _lsm_tc(q):
    _, _, m2 = q.shape
    n = N
    return pl.pallas_call(
        _lsm_body,
        grid=(n // _BN,),
        in_specs=[pl.BlockSpec((NC, _BN, m2), lambda i: (0, i, 0))],
        out_specs=pl.BlockSpec((_BN, NC * m2), lambda i: (i, 0)),
        out_shape=jax.ShapeDtypeStruct((n, NC * m2), jnp.float32),
    )(q)


def _pack_words(a_bf16):
    rows, d = a_bf16.shape
    return jax.lax.bitcast_convert_type(
        a_bf16.reshape(rows, d // 2, 2), jnp.int32)


def kernel(x, adj_indices, adj_values, W1, W2):
    row = adj_indices[0]
    col = adj_indices[1]

    pad = NS * EW - E
    colp = jnp.concatenate([col, jnp.zeros((pad,), jnp.int32)]).reshape(NS, NCH, C)
    rowp = jnp.concatenate([row, jnp.zeros((pad,), jnp.int32)]).reshape(NS, NCH, C)
    valp = jnp.concatenate(
        [adj_values, jnp.zeros((pad,), jnp.float32)]).reshape(NS, NCH, C)

    p1c = _packed_perm(64)
    w1s = jnp.stack([W1[:, :64][:, p1c], W1[:, 64:][:, p1c]])
    p2c = _packed_perm(32)
    w2s = jnp.stack([W2[:, :32][:, p2c], W2[:, 32:][:, p2c]])

    support1 = _pack_words(_matmul_split_tc(x, w1s))
    p1 = _spmm_sc(support1, colp, rowp, valp, 64)
    support2 = _pack_words(_fuse2_tc(p1, w2s))
    p2 = _spmm_sc(support2, colp, rowp, valp, 32)
    return _lsm_tc(p2)


# speedup vs baseline: 1.0960x; 1.0960x over previous
import functools

import jax
import jax.numpy as jnp
import numpy as np
from jax import lax
from jax.experimental import pallas as pl
from jax.experimental.pallas import tpu as pltpu
from jax.experimental.pallas import tpu_sc as plsc

N = 10000
E = 320000
NC = 2
NS = 16
C = 128
EW = -(-E // (NS * 2 * C)) * 2 * C
NCH = EW // C
NP = 10240
ROWS_PER_TILE = NP // NS
ZROWS = 128

_PERM32 = np.empty(32, dtype=np.int32)
_PERM32[0::2] = np.arange(16)
_PERM32[1::2] = np.arange(16) + 16


def _packed_perm(d):
    return (np.arange(0, d, 32)[:, None] + _PERM32[None, :]).reshape(d)


def _spmm_sc(dense2w, colp, rowp, valp, d2):
    d2b = d2 // 2
    mesh = plsc.VectorSubcoreMesh(core_axis_name="c", subcore_axis_name="s")

    @functools.partial(
        pl.kernel,
        out_type=jax.ShapeDtypeStruct((NC, NP, d2), jnp.float32),
        mesh=mesh,
        scratch_types=[
            pltpu.VMEM((NCH, C), jnp.int32),
            pltpu.VMEM((NCH, C), jnp.int32),
            pltpu.VMEM((NCH, C), jnp.float32),
            pltpu.VMEM((C, d2), jnp.bfloat16),
            pltpu.VMEM((C, d2), jnp.bfloat16),
            pltpu.VMEM((C, d2), jnp.float32),
            pltpu.VMEM((C, d2), jnp.float32),
            pltpu.VMEM_SHARED((NP, d2), jnp.float32),
            pltpu.SemaphoreType.DMA,
            pltpu.SemaphoreType.DMA,
            pltpu.SemaphoreType.DMA,
            pltpu.SemaphoreType.DMA,
        ],
        compiler_params=pltpu.CompilerParams(use_tc_tiling_on_sc=False, needs_layout_passes=False),
    )
    def k(dense_hbm, col_hbm, row_hbm, val_hbm, out_hbm,
          colv, rowv, valv, gbuf0, gbuf1, fbuf0, fbuf1, acc,
          gsem0, gsem1, ssem0, ssem1):
        cid = lax.axis_index("c")
        sid = lax.axis_index("s")
        gbufs = (gbuf0, gbuf1)
        fbufs = (fbuf0, fbuf1)
        gsems = (gsem0, gsem1)
        ssems = (ssem0, ssem1)

        pltpu.sync_copy(col_hbm.at[sid], colv)
        pltpu.sync_copy(row_hbm.at[sid], rowv)
        pltpu.sync_copy(val_hbm.at[sid], valv)

        off = cid * N

        def shift(j, c2):
            for kk in range(C // 16):
                sl = pl.ds(kk * 16, 16)
                colv[j, sl] = colv[j, sl] + off
            return c2

        lax.fori_loop(0, NCH, shift, 0)

        zero = jnp.zeros((16,), jnp.float32)

        def zrow(i, carry):
            for kk in range(d2 // 16):
                fbuf0[i, pl.ds(kk * 16, 16)] = zero
            return carry

        lax.fori_loop(0, C, zrow, 0)
        for b in range(ROWS_PER_TILE // C):
            pltpu.sync_copy(
                fbuf0, acc.at[pl.ds(sid * ROWS_PER_TILE + b * C, C)])
        plsc.subcore_barrier()

        hi_mask = jnp.full((16,), -65536, jnp.int32)

        def scale(gb, fb, j):
            def grp(t, c2):
                v16 = valv[j, pl.ds(t * 16, 16)]
                for e16 in range(16):
                    v = v16[e16]
                    e = t * 16 + e16
                    for kk in range(d2b // 16):
                        w = plsc.bitcast(gb[e, pl.ds(kk * 32, 32)], jnp.int32)
                        lo = plsc.bitcast(w << 16, jnp.float32)
                        hi = plsc.bitcast(w & hi_mask, jnp.float32)
                        fb[e, pl.ds(kk * 32, 16)] = lo * v
                        fb[e, pl.ds(kk * 32 + 16, 16)] = hi * v
                return c2

            lax.fori_loop(0, C // 16, grp, 0)

        pltpu.async_copy(dense_hbm.at[colv.at[0]], gbuf0, gsem0)
        pltpu.async_copy(dense_hbm.at[colv.at[1]], gbuf1, gsem1)

        def body(jj, carry):
            for s in range(2):
                j = 2 * jj + s
                pltpu.make_async_copy(
                    dense_hbm.at[colv.at[j]], gbufs[s], gsems[s]).wait()

                @pl.when(jj >= 1)
                def _wait_scatter():
                    pltpu.make_async_copy(
                        fbufs[s], acc.at[rowv.at[j - 2]], ssems[s]).wait()

                scale(gbufs[s], fbufs[s], j)

                @pl.when(jj < NCH // 2 - 1)
                def _next_gather():
                    pltpu.async_copy(
                        dense_hbm.at[colv.at[j + 2]], gbufs[s], gsems[s])

                pltpu.async_copy(
                    fbufs[s], acc.at[rowv.at[j]], ssems[s], add=True)
            return carry

        lax.fori_loop(0, NCH // 2, body, 0)
        pltpu.make_async_copy(fbuf0, acc.at[rowv.at[NCH - 2]], ssem0).wait()
        pltpu.make_async_copy(fbuf1, acc.at[rowv.at[NCH - 1]], ssem1).wait()

        plsc.subcore_barrier()
        pltpu.sync_copy(acc.at[pl.ds(sid * ROWS_PER_TILE, ROWS_PER_TILE)],
                        out_hbm.at[cid, pl.ds(sid * ROWS_PER_TILE, ROWS_PER_TILE)])

    return k(dense2w, colp, rowp, valp)


_BN = 1000


def _mm_body(x_ref, w_ref, o_ref):
    o_ref[...] = jnp.dot(x_ref[...], w_ref[0],
                         preferred_element_type=jnp.float32
                         ).astype(jnp.bfloat16)


def _matmul_split_tc(x, ws):
    n, kd = x.shape
    m2 = ws.shape[2]
    return pl.pallas_call(
        _mm_body,
        grid=(NC, n // _BN),
        in_specs=[
            pl.BlockSpec((_BN, kd), lambda c, i: (i, 0)),
            pl.BlockSpec((1, kd, m2), lambda c, i: (c, 0, 0)),
        ],
        out_specs=pl.BlockSpec(
            (_BN, m2), lambda c, i: (c * (n // _BN) + i, 0)),
        out_shape=jax.ShapeDtypeStruct((NC * n, m2), jnp.bfloat16),
    )(x, ws)


def _fuse2_body(p_ref, w_ref, o_ref):
    h0 = jnp.maximum(p_ref[0], 0.0)
    h1 = jnp.maximum(p_ref[1], 0.0)
    w = w_ref[0]
    kd2 = p_ref.shape[2]
    o_ref[...] = (
        jnp.dot(h0, w[:kd2], preferred_element_type=jnp.float32)
        + jnp.dot(h1, w[kd2:], preferred_element_type=jnp.float32)
    ).astype(jnp.bfloat16)


def _fuse2_tc(p, ws):
    _, _, kd2 = p.shape
    m2 = ws.shape[2]
    n = N
    return pl.pallas_call(
        _fuse2_body,
        grid=(NC, n // _BN),
        in_specs=[
            pl.BlockSpec((NC, _BN, kd2), lambda c, i: (0, i, 0)),
            pl.BlockSpec((1, 2 * kd2, m2), lambda c, i: (c, 0, 0)),
        ],
        out_specs=pl.BlockSpec(
            (_BN, m2), lambda c, i: (c * (n // _BN) + i, 0)),
        out_shape=jax.ShapeDtypeStruct((NC * n, m2), jnp.bfloat16),
    )(p, ws)


def _lsm_body(q_ref, o_ref):
    q0 = q_ref[0]
    q1 = q_ref[1]
    m = jnp.maximum(jnp.max(q0, axis=1, keepdims=True),
                    jnp.max(q1, axis=1, keepdims=True))
    ssum = (jnp.sum(jnp.exp(q0 - m), axis=1, keepdims=True)
            + jnp.sum(jnp.exp(q1 - m), axis=1, keepdims=True))
    lse = jnp.log(ssum) + m
    m2 = q0.shape[1]
    o_ref[:, :m2] = q0 - lse
    o_ref[:, m2:] = q1 - lse


def _lsm_tc(q):
    _, _, m2 = q.shape
    n = N
    return pl.pallas_call(
        _lsm_body,
        grid=(n // _BN,),
        in_specs=[pl.BlockSpec((NC, _BN, m2), lambda i: (0, i, 0))],
        out_specs=pl.BlockSpec((_BN, NC * m2), lambda i: (i, 0)),
        out_shape=jax.ShapeDtypeStruct((n, NC * m2), jnp.float32),
    )(q)


def kernel(x, adj_indices, adj_values, W1, W2):
    row = adj_indices[0]
    col = adj_indices[1]

    pad = NS * EW - E
    colp = jnp.concatenate([col, jnp.zeros((pad,), jnp.int32)]).reshape(NS, NCH, C)
    rowp = jnp.concatenate([row, jnp.zeros((pad,), jnp.int32)]).reshape(NS, NCH, C)
    valp = jnp.concatenate(
        [adj_values, jnp.zeros((pad,), jnp.float32)]).reshape(NS, NCH, C)

    p1c = _packed_perm(64)
    w1s = jnp.stack([W1[:, :64][:, p1c], W1[:, 64:][:, p1c]])
    p2c = _packed_perm(32)
    w2s = jnp.stack([W2[:, :32][:, p2c], W2[:, 32:][:, p2c]])

    support1 = _matmul_split_tc(x, w1s)
    p1 = _spmm_sc(support1, colp, rowp, valp, 64)
    support2 = _fuse2_tc(p1, w2s)
    p2 = _spmm_sc(support2, colp, rowp, valp, 32)
    return _lsm_tc(p2)
